# baseline (speedup 1.0000x reference)
import functools

import jax
import jax.numpy as jnp
from jax import lax
from jax.experimental import pallas as pl
from jax.experimental.pallas import tpu as pltpu
from jax.experimental.pallas import tpu_sc as plsc

_B = 16384
_D = 32
_NNZ = 327680
_L = 128
_NROWS = _NNZ // _L
_NC, _NS = 2, 16
_NW = _NC * _NS
_RPW = _NROWS // _NW
_CHUNK = 8
_NCHUNK = _RPW // _CHUNK
_RPT = _B // _NS


def _sc_column(table, fidx2d, ridx2d, wv2d):
    mesh = plsc.VectorSubcoreMesh(core_axis_name="c", subcore_axis_name="s")

    @functools.partial(
        pl.kernel,
        out_type=jax.ShapeDtypeStruct((_NC, _B, _D), jnp.float32),
        mesh=mesh,
        scratch_types=[
            pltpu.VMEM_SHARED((_B, _D), jnp.float32),
            pltpu.VMEM((2, _CHUNK, _L, _D), jnp.float32),
            pltpu.VMEM((_RPW, _L), jnp.int32),
            pltpu.VMEM((_RPW, _L), jnp.int32),
            pltpu.VMEM((_RPW, _L), jnp.float32),
            pltpu.SemaphoreType.DMA,
            pltpu.SemaphoreType.DMA,
            pltpu.SemaphoreType.DMA,
            pltpu.SemaphoreType.DMA,
        ],
        compiler_params=pltpu.CompilerParams(use_tc_tiling_on_sc=False),
    )
    def k(tbl, fsrc, rsrc, wsrc, out, acc, rows, fidx, ridx, wv,
          gsem0, gsem1, ssem0, ssem1):
        s = lax.axis_index("s")
        c = lax.axis_index("c")
        wid = s * _NC + c
        rb = wid * _RPW
        gsem = (gsem0, gsem1)
        ssem = (ssem0, ssem1)

        @pl.loop(0, _L)
        def _(i):
            z = jnp.zeros((16,), jnp.float32)
            rows[0, 0, i, pl.ds(0, 16)] = z
            rows[0, 0, i, pl.ds(16, 16)] = z

        @pl.loop(0, _RPT // _L)
        def _(jj):
            pltpu.sync_copy(rows.at[0, 0],
                            acc.at[pl.ds(s * _RPT + jj * _L, _L)])

        pltpu.sync_copy(fsrc.at[pl.ds(rb, _RPW)], fidx)
        pltpu.sync_copy(rsrc.at[pl.ds(rb, _RPW)], ridx)
        pltpu.sync_copy(wsrc.at[pl.ds(rb, _RPW)], wv)

        plsc.subcore_barrier()

        def fire_gathers(kk, b):
            return [
                pltpu.async_copy(tbl.at[fidx.at[kk * _CHUNK + j]],
                                 rows.at[b, j], gsem[b])
                for j in range(_CHUNK)
            ]

        def weight_chunk(kk, b):
            @pl.loop(0, _CHUNK)
            def _(j):
                @pl.loop(0, _L // 16)
                def _(g):
                    wvec = wv[kk * _CHUNK + j, pl.ds(g * 16, 16)]
                    for t in range(16):
                        wgt = wvec[t]
                        i = g * 16 + t
                        lo = rows[b, j, i, pl.ds(0, 16)]
                        hi = rows[b, j, i, pl.ds(16, 16)]
                        rows[b, j, i, pl.ds(0, 16)] = lo * wgt
                        rows[b, j, i, pl.ds(16, 16)] = hi * wgt

        def fire_scatters(kk, b):
            return [
                pltpu.async_copy(rows.at[b, j],
                                 acc.at[ridx.at[kk * _CHUNK + j]],
                                 ssem[b], add=True)
                for j in range(_CHUNK)
            ]

        gd = [None, None]
        sd = [None, None]
        gd[0] = fire_gathers(0, 0)
        for kk in range(1, _NCHUNK + 1):
            b = kk % 2
            pb = 1 - b
            if sd[b] is not None:
                for d in sd[b]:
                    d.wait()
                sd[b] = None
            if kk < _NCHUNK:
                gd[b] = fire_gathers(kk, b)
            for d in gd[pb]:
                d.wait()
            weight_chunk(kk - 1, pb)
            sd[pb] = fire_scatters(kk - 1, pb)
        for b in range(2):
            if sd[b] is not None:
                for d in sd[b]:
                    d.wait()
                sd[b] = None

        plsc.subcore_barrier()
        pltpu.sync_copy(
            acc.at[pl.ds(s * _RPT, _RPT)],
            out.at[c, pl.ds(s * _RPT, _RPT)],
        )

    return k(table, fidx2d, ridx2d, wv2d)


_TB = 4096


def _tc_relayout(table, after=None):
    V, _ = table.shape
    tt = table.T
    VB = 4 * _TB
    grid = (V + VB - 1) // VB

    if after is None:
        def body(i_ref, o_ref):
            x = i_ref[...]
            x4 = jnp.concatenate(
                [x[:, _TB * a:_TB * (a + 1)] for a in range(4)], axis=0)
            o_ref[...] = x4.T

        args, in_specs = (tt,), [pl.BlockSpec((32, VB), lambda i: (0, i))]
    else:
        n_taint = len(after)

        def body(i_ref, *rest):
            o_ref = rest[n_taint]
            x = i_ref[...]
            x4 = jnp.concatenate(
                [x[:, _TB * a:_TB * (a + 1)] for a in range(4)], axis=0)
            o_ref[...] = x4.T

        args = (tt, *[a[:8, :32] for a in after])
        in_specs = [pl.BlockSpec((32, VB), lambda i: (0, i))] + [
            pl.BlockSpec((8, 32), lambda i: (0, 0)) for _ in after]

    out = pl.pallas_call(
        body,
        out_shape=jax.ShapeDtypeStruct((grid * _TB, 128), jnp.float32),
        grid=(grid,),
        in_specs=in_specs,
        out_specs=pl.BlockSpec((_TB, 128), lambda i: (i, 0)),
    )(*args)
    return out.reshape(grid * VB, 32)


def _remap_ids(feat_ids):
    f = feat_ids.astype(jnp.int32)
    return (f // (4 * _TB)) * (4 * _TB) + (f % _TB) * 4 + (f % (4 * _TB)) // _TB


def _tc_combine(p0, p1):
    a = p0.reshape(_NC, (_B * _D) // 128, 128)
    b = p1.reshape(_NC, (_B * _D) // 128, 128)

    def body(a_ref, b_ref, o_ref):
        o_ref[...] = (a_ref[0] + a_ref[1]) + (b_ref[0] + b_ref[1])

    out = pl.pallas_call(
        body,
        out_shape=jax.ShapeDtypeStruct(((_B * _D) // 128, 128), jnp.float32),
        grid=(4,),
        in_specs=[pl.BlockSpec((_NC, 1024, 128), lambda i: (0, i, 0)),
                  pl.BlockSpec((_NC, 1024, 128), lambda i: (0, i, 0))],
        out_specs=pl.BlockSpec((1024, 128), lambda i: (i, 0)),
    )(a, b)
    return out.reshape(_B, _D)


def kernel(row_ids0, feat_ids0, weights0, row_ids1, feat_ids1, weights1,
           table0, table1):
    f0 = _remap_ids(feat_ids0).reshape(_NROWS, _L)
    r0 = row_ids0.astype(jnp.int32).reshape(_NROWS, _L)
    w0 = weights0.reshape(_NROWS, _L)
    f1 = _remap_ids(feat_ids1).reshape(_NROWS, _L)
    r1 = row_ids1.astype(jnp.int32).reshape(_NROWS, _L)
    w1 = weights1.reshape(_NROWS, _L)
    t1 = _tc_relayout(table1)
    p1 = _sc_column(t1, f1, r1, w1)
    t0 = _tc_relayout(table0, after=(t1, f0, r0))
    p0 = _sc_column(t0, f0, r0, w0)
    return _tc_combine(p0, p1)


# speedup vs baseline: 1.0383x; 1.0383x over previous
import functools

import jax
import jax.numpy as jnp
from jax import lax
from jax.experimental import pallas as pl
from jax.experimental.pallas import tpu as pltpu
from jax.experimental.pallas import tpu_sc as plsc

_B = 16384
_D = 32
_NNZ = 327680
_L = 128
_NROWS = _NNZ // _L
_NC, _NS = 2, 16
_NW = _NC * _NS
_RPW = _NROWS // _NW
_CHUNK = 8
_NCHUNK = _RPW // _CHUNK
_RPT = _B // _NS


def _sc_column(table, fidx2d, ridx2d, wv2d):
    mesh = plsc.VectorSubcoreMesh(core_axis_name="c", subcore_axis_name="s")

    @functools.partial(
        pl.kernel,
        out_type=jax.ShapeDtypeStruct((_NC, _B, _D), jnp.float32),
        mesh=mesh,
        scratch_types=[
            pltpu.VMEM_SHARED((_B, _D), jnp.float32),
            pltpu.VMEM((2, _CHUNK, _L, _D), jnp.float32),
            pltpu.VMEM((_RPW, _L), jnp.int32),
            pltpu.VMEM((_RPW, _L), jnp.int32),
            pltpu.VMEM((_RPW, _L), jnp.float32),
            pltpu.SemaphoreType.DMA,
            pltpu.SemaphoreType.DMA,
            pltpu.SemaphoreType.DMA,
            pltpu.SemaphoreType.DMA,
        ],
        compiler_params=pltpu.CompilerParams(use_tc_tiling_on_sc=False),
    )
    def k(tbl, fsrc, rsrc, wsrc, out, acc, rows, fidx, ridx, wv,
          gsem0, gsem1, ssem0, ssem1):
        s = lax.axis_index("s")
        c = lax.axis_index("c")
        wid = s * _NC + c
        rb = wid * _RPW
        gsem = (gsem0, gsem1)
        ssem = (ssem0, ssem1)

        @pl.loop(0, _L)
        def _(i):
            z = jnp.zeros((16,), jnp.float32)
            rows[0, 0, i, pl.ds(0, 16)] = z
            rows[0, 0, i, pl.ds(16, 16)] = z

        @pl.loop(0, _RPT // _L)
        def _(jj):
            pltpu.sync_copy(rows.at[0, 0],
                            acc.at[pl.ds(s * _RPT + jj * _L, _L)])

        pltpu.sync_copy(fsrc.at[pl.ds(rb, _RPW)], fidx)
        pltpu.sync_copy(rsrc.at[pl.ds(rb, _RPW)], ridx)
        pltpu.sync_copy(wsrc.at[pl.ds(rb, _RPW)], wv)

        plsc.subcore_barrier()

        def fire_gathers(kk, b):
            return [
                pltpu.async_copy(tbl.at[fidx.at[kk * _CHUNK + j]],
                                 rows.at[b, j], gsem[b])
                for j in range(_CHUNK)
            ]

        def weight_chunk(kk, b):
            @pl.loop(0, _CHUNK)
            def _(j):
                @pl.loop(0, _L // 16)
                def _(g):
                    wvec = wv[kk * _CHUNK + j, pl.ds(g * 16, 16)]
                    for t in range(16):
                        wgt = wvec[t]
                        i = g * 16 + t
                        lo = rows[b, j, i, pl.ds(0, 16)]
                        hi = rows[b, j, i, pl.ds(16, 16)]
                        rows[b, j, i, pl.ds(0, 16)] = lo * wgt
                        rows[b, j, i, pl.ds(16, 16)] = hi * wgt

        def scatter_chunk(kk, b):
            for j in range(_CHUNK):
                pltpu.sync_copy(rows.at[b, j],
                                acc.at[ridx.at[kk * _CHUNK + j]],
                                add=True)

        gd = [None, None]
        gd[0] = fire_gathers(0, 0)
        for kk in range(1, _NCHUNK + 1):
            b = kk % 2
            pb = 1 - b
            if kk < _NCHUNK:
                gd[b] = fire_gathers(kk, b)
            for d in gd[pb]:
                d.wait()
            weight_chunk(kk - 1, pb)
            scatter_chunk(kk - 1, pb)

        plsc.subcore_barrier()
        pltpu.sync_copy(
            acc.at[pl.ds(s * _RPT, _RPT)],
            out.at[c, pl.ds(s * _RPT, _RPT)],
        )

    return k(table, fidx2d, ridx2d, wv2d)


_TB = 8192


def _tc_relayout(table, after=None):
    V, _ = table.shape
    tt = table.T
    VB = 4 * _TB
    grid = (V + VB - 1) // VB

    if after is None:
        def body(i_ref, o_ref):
            x = i_ref[...]
            x4 = jnp.concatenate(
                [x[:, _TB * a:_TB * (a + 1)] for a in range(4)], axis=0)
            o_ref[...] = x4.T

        args, in_specs = (tt,), [pl.BlockSpec((32, VB), lambda i: (0, i))]
    else:
        n_taint = len(after)

        def body(i_ref, *rest):
            o_ref = rest[n_taint]
            x = i_ref[...]
            x4 = jnp.concatenate(
                [x[:, _TB * a:_TB * (a + 1)] for a in range(4)], axis=0)
            o_ref[...] = x4.T

        args = (tt, *[a[:8, :32] for a in after])
        in_specs = [pl.BlockSpec((32, VB), lambda i: (0, i))] + [
            pl.BlockSpec((8, 32), lambda i: (0, 0)) for _ in after]

    out = pl.pallas_call(
        body,
        out_shape=jax.ShapeDtypeStruct((grid * _TB, 128), jnp.float32),
        grid=(grid,),
        in_specs=in_specs,
        out_specs=pl.BlockSpec((_TB, 128), lambda i: (i, 0)),
    )(*args)
    return out.reshape(grid * VB, 32)


def _remap_ids(feat_ids):
    f = feat_ids.astype(jnp.int32)
    return (f // (4 * _TB)) * (4 * _TB) + (f % _TB) * 4 + (f % (4 * _TB)) // _TB


def _tc_combine(p0, p1):
    a = p0.reshape(_NC, (_B * _D) // 128, 128)
    b = p1.reshape(_NC, (_B * _D) // 128, 128)

    def body(a_ref, b_ref, o_ref):
        o_ref[...] = (a_ref[0] + a_ref[1]) + (b_ref[0] + b_ref[1])

    out = pl.pallas_call(
        body,
        out_shape=jax.ShapeDtypeStruct(((_B * _D) // 128, 128), jnp.float32),
        grid=(4,),
        in_specs=[pl.BlockSpec((_NC, 1024, 128), lambda i: (0, i, 0)),
                  pl.BlockSpec((_NC, 1024, 128), lambda i: (0, i, 0))],
        out_specs=pl.BlockSpec((1024, 128), lambda i: (i, 0)),
    )(a, b)
    return out.reshape(_B, _D)


def kernel(row_ids0, feat_ids0, weights0, row_ids1, feat_ids1, weights1,
           table0, table1):
    f0 = _remap_ids(feat_ids0).reshape(_NROWS, _L)
    r0 = row_ids0.astype(jnp.int32).reshape(_NROWS, _L)
    w0 = weights0.reshape(_NROWS, _L)
    f1 = _remap_ids(feat_ids1).reshape(_NROWS, _L)
    r1 = row_ids1.astype(jnp.int32).reshape(_NROWS, _L)
    w1 = weights1.reshape(_NROWS, _L)
    t1 = _tc_relayout(table1)
    p1 = _sc_column(t1, f1, r1, w1)
    t0 = _tc_relayout(table0, after=(t1, f0, r0))
    p0 = _sc_column(t0, f0, r0, w0)
    return _tc_combine(p0, p1)
